# Initial kernel scaffold; baseline (speedup 1.0000x reference)
import jax
import jax.numpy as jnp
from jax.experimental import pallas as pl


def kernel(x, pos_embedding):
    raise NotImplementedError("write your pallas kernel here")


# speedup vs baseline: 1.4977x; 1.4977x over previous
import jax
import jax.numpy as jnp
from jax.experimental import pallas as pl


_CHUNK = 512


def _add_body(x_ref, pos_ref, out_ref):
    out_ref[0] = x_ref[0] + pos_ref[...]


def kernel(x, pos_embedding):
    batch, seq, dim = x.shape
    grid = (seq // _CHUNK, batch)
    return pl.pallas_call(
        _add_body,
        grid=grid,
        in_specs=[
            pl.BlockSpec((1, _CHUNK, dim), lambda i, b: (b, i, 0)),
            pl.BlockSpec((_CHUNK, dim), lambda i, b: (i, 0)),
        ],
        out_specs=pl.BlockSpec((1, _CHUNK, dim), lambda i, b: (b, i, 0)),
        out_shape=jax.ShapeDtypeStruct((batch, seq, dim), x.dtype),
    )(x, pos_embedding)
